# Initial kernel scaffold; baseline (speedup 1.0000x reference)
import jax
import jax.numpy as jnp
from jax.experimental import pallas as pl


def kernel(track_point_feats, p2t_src, p2t_dst, W, b, ln_gamma, ln_beta):
    raise NotImplementedError("write your pallas kernel here")


# speedup vs baseline: 1.1100x; 1.1100x over previous
import jax
import jax.numpy as jnp
from jax.experimental import pallas as pl
from jax.experimental.pallas import tpu as pltpu

N_POINTS = 10000
N_TRACKS = 1000
N_EDGES = 320000
D = 128
LN_EPS = 1e-5


def _mlp_body(x_ref, w_ref, b_ref, g_ref, be_ref, o_ref):
    h = jnp.dot(x_ref[...], w_ref[...],
                preferred_element_type=jnp.float32,
                precision=jax.lax.Precision.HIGHEST) + b_ref[...]
    mu = jnp.mean(h, axis=-1, keepdims=True)
    var = jnp.mean((h - mu) ** 2, axis=-1, keepdims=True)
    hn = (h - mu) * jax.lax.rsqrt(var + LN_EPS)
    o_ref[...] = jnp.maximum(hn * g_ref[...] + be_ref[...], 0.0)


def _mlp(x, W, b, gamma, beta):
    return pl.pallas_call(
        _mlp_body,
        out_shape=jax.ShapeDtypeStruct((N_POINTS, D), jnp.float32),
        grid=(5,),
        in_specs=[
            pl.BlockSpec((2000, D), lambda i: (i, 0)),
            pl.BlockSpec((D, D), lambda i: (0, 0)),
            pl.BlockSpec((1, D), lambda i: (0, 0)),
            pl.BlockSpec((1, D), lambda i: (0, 0)),
            pl.BlockSpec((1, D), lambda i: (0, 0)),
        ],
        out_specs=pl.BlockSpec((2000, D), lambda i: (i, 0)),
    )(x, W, b.reshape(1, D), gamma.reshape(1, D), beta.reshape(1, D))


def kernel(track_point_feats, p2t_src, p2t_dst, W, b, ln_gamma, ln_beta):
    x = track_point_feats
    h = _mlp(x, W, b, ln_gamma, ln_beta)
    track_pool = jax.ops.segment_max(h[p2t_src], p2t_dst, num_segments=N_TRACKS)
    track_pool = jnp.where(jnp.isneginf(track_pool), 0.0, track_pool)
    point_pool = jax.ops.segment_sum(track_pool[p2t_dst], p2t_src,
                                     num_segments=N_POINTS)
    out_features = jnp.concatenate([x, point_pool], axis=1)
    return out_features, track_pool


# speedup vs baseline: 2.5758x; 2.3204x over previous
import functools

import jax
import jax.numpy as jnp
from jax import lax
from jax.experimental import pallas as pl
from jax.experimental.pallas import tpu as pltpu
from jax.experimental.pallas import tpu_sc as plsc

N_POINTS = 10000
N_TRACKS = 1000
N_EDGES = 320000
D = 128
HALF = 64
LN_EPS = 1e-5

NC = 2
NS = 16

_SC_MESH = plsc.VectorSubcoreMesh(core_axis_name="c", subcore_axis_name="s")


def _mlp_body(x_ref, w_ref, b_ref, g_ref, be_ref, o_ref):
    h = jnp.dot(x_ref[...], w_ref[...], preferred_element_type=jnp.float32,
                precision=jax.lax.Precision.HIGHEST) + b_ref[...]
    mu = jnp.mean(h, axis=-1, keepdims=True)
    var = jnp.mean((h - mu) ** 2, axis=-1, keepdims=True)
    hn = (h - mu) * jax.lax.rsqrt(var + LN_EPS)
    o_ref[...] = jnp.maximum(hn * g_ref[...] + be_ref[...], 0.0)


def _mlp(x, W, b, gamma, beta):
    return pl.pallas_call(
        _mlp_body,
        out_shape=jax.ShapeDtypeStruct((N_POINTS, D), jnp.float32),
        grid=(5,),
        in_specs=[
            pl.BlockSpec((2000, D), lambda i: (i, 0)),
            pl.BlockSpec((D, D), lambda i: (0, 0)),
            pl.BlockSpec((1, D), lambda i: (0, 0)),
            pl.BlockSpec((1, D), lambda i: (0, 0)),
            pl.BlockSpec((1, D), lambda i: (0, 0)),
        ],
        out_specs=pl.BlockSpec((2000, D), lambda i: (i, 0)),
    )(x, W, b.reshape(1, D), gamma.reshape(1, D), beta.reshape(1, D))


P1_CHUNK = 80
P1_GROUPS = 8
P1_EPT = N_EDGES // (NC * P1_GROUPS)
P1_NCHUNK = P1_EPT // P1_CHUNK
TRK_SLICE = N_TRACKS // P1_GROUPS


def _seg_max(h, src, dst, zacc):
    @functools.partial(
        pl.kernel,
        out_type=jax.ShapeDtypeStruct((NC * 2 * N_TRACKS * HALF,),
                                      jnp.float32),
        mesh=_SC_MESH,
        scratch_types=[
            pltpu.VMEM((N_TRACKS * HALF,), jnp.float32),
            pltpu.VMEM((P1_CHUNK,), jnp.int32),
            pltpu.VMEM((P1_CHUNK,), jnp.int32),
            pltpu.VMEM((P1_CHUNK, D), jnp.float32),
            pltpu.VMEM((N_TRACKS * HALF // 8,), jnp.float32),
            pltpu.VMEM_SHARED((8 * N_TRACKS * HALF,), jnp.float32),
        ],
    )
    def body(h_hbm, src_hbm, dst_hbm, z_hbm, out_hbm,
             acc, sbuf, dbuf, rbuf, tmp, stage):
        k = lax.axis_index("c")
        s = lax.axis_index("s")
        half = s // P1_GROUPS
        grp = s % P1_GROUPS
        base = k * (N_EDGES // NC) + grp * P1_EPT

        pltpu.sync_copy(z_hbm, acc)

        def edge_loop(coff):
            @pl.loop(0, P1_NCHUNK)
            def _chunk(t):
                off = base + t * P1_CHUNK
                pltpu.sync_copy(src_hbm.at[pl.ds(off, P1_CHUNK)], sbuf)
                pltpu.sync_copy(dst_hbm.at[pl.ds(off, P1_CHUNK)], dbuf)
                pltpu.sync_copy(h_hbm.at[sbuf], rbuf)

                @pl.loop(0, P1_CHUNK // 16)
                def _edge16(i):
                    dvec = dbuf[pl.ds(i * 16, 16)]
                    for jj in range(16):
                        dd = dvec[jj]
                        j = i * 16 + jj
                        for cc in range(HALF // 16):
                            asl = pl.ds(dd * HALF + cc * 16, 16)
                            rsl = pl.ds(coff + cc * 16, 16)
                            acc[asl] = jnp.maximum(acc[asl], rbuf[j, rsl])

        @pl.when(half == 0)
        def _lo():
            edge_loop(0)

        @pl.when(half == 1)
        def _hi():
            edge_loop(HALF)

        FLAT = N_TRACKS * HALF
        CNK = FLAT // 8
        for m in (4, 2, 1):
            @pl.when(jnp.logical_and(grp >= m, grp < 2 * m))
            def _stage():
                slot = half * 4 + (grp - m)
                pltpu.sync_copy(acc, stage.at[pl.ds(slot * FLAT, FLAT)])

            plsc.subcore_barrier()

            @pl.when(grp < m)
            def _merge():
                slot = half * 4 + grp

                @pl.loop(0, 8)
                def _cnk(c):
                    pltpu.sync_copy(
                        stage.at[pl.ds(slot * FLAT + c * CNK, CNK)], tmp)

                    @pl.loop(0, CNK // 16)
                    def _vec(v):
                        asl = pl.ds(c * CNK + v * 16, 16)
                        tsl = pl.ds(v * 16, 16)
                        acc[asl] = jnp.maximum(acc[asl], tmp[tsl])

            plsc.subcore_barrier()

        @pl.when(grp == 0)
        def _writeout():
            pltpu.sync_copy(
                acc, out_hbm.at[pl.ds((k * 2 + half) * FLAT, FLAT)])

    return body(h, src, dst, zacc)


def _tpmax_body(p_ref, o_ref):
    m = jnp.maximum(p_ref[0], p_ref[1])
    o_ref[...] = jnp.concatenate([m[0], m[1]], axis=1)


def _tpmax(tp_part):
    return pl.pallas_call(
        _tpmax_body,
        out_shape=jax.ShapeDtypeStruct((N_TRACKS, D), jnp.float32),
        grid=(1,),
        in_specs=[pl.BlockSpec((NC, 2, N_TRACKS, HALF),
                               lambda i: (0, 0, 0, 0))],
        out_specs=pl.BlockSpec((N_TRACKS, D), lambda i: (0, 0)),
    )(tp_part)


P2_CHUNK = 80
P2_EPT = N_EDGES // (NC * NS)
P2_NCHUNK = P2_EPT // P2_CHUNK
PP_SLICE = N_POINTS // NS


def _seg_sum(tp, src, dst, zpp):
    @functools.partial(
        pl.kernel,
        out_type=jax.ShapeDtypeStruct((NC, NS, PP_SLICE, D), jnp.float32),
        mesh=_SC_MESH,
        scratch_types=[
            pltpu.VMEM((P2_CHUNK,), jnp.int32),
            pltpu.VMEM((P2_CHUNK,), jnp.int32),
            pltpu.VMEM((P2_CHUNK, D), jnp.float32),
            pltpu.VMEM_SHARED((N_POINTS, D), jnp.float32),
        ],
    )
    def body(tp_hbm, src_hbm, dst_hbm, z_hbm, out_hbm, sbuf, dbuf, rbuf, pp):
        k = lax.axis_index("c")
        s = lax.axis_index("s")
        base = k * (N_EDGES // NC) + s * P2_EPT
        rowbase = s * PP_SLICE

        pltpu.sync_copy(z_hbm.at[s], pp.at[pl.ds(rowbase, PP_SLICE)])
        plsc.subcore_barrier()

        @pl.loop(0, P2_NCHUNK)
        def _chunk(t):
            off = base + t * P2_CHUNK
            pltpu.sync_copy(src_hbm.at[pl.ds(off, P2_CHUNK)], sbuf)
            pltpu.sync_copy(dst_hbm.at[pl.ds(off, P2_CHUNK)], dbuf)
            pltpu.sync_copy(tp_hbm.at[dbuf], rbuf)
            pltpu.sync_copy(rbuf, pp.at[sbuf], add=True)

        plsc.subcore_barrier()
        pltpu.sync_copy(pp.at[pl.ds(rowbase, PP_SLICE)], out_hbm.at[k, s])

    return body(tp, src, dst, zpp)


def _out_body(x_ref, pp_ref, o_ref):
    o_ref[...] = jnp.concatenate([x_ref[...], pp_ref[0] + pp_ref[1]], axis=1)


def _outk(x, pp_part):
    return pl.pallas_call(
        _out_body,
        out_shape=jax.ShapeDtypeStruct((N_POINTS, 2 * D), jnp.float32),
        grid=(5,),
        in_specs=[
            pl.BlockSpec((2000, D), lambda i: (i, 0)),
            pl.BlockSpec((NC, 2000, D), lambda i: (0, i, 0)),
        ],
        out_specs=pl.BlockSpec((2000, 2 * D), lambda i: (i, 0)),
    )(x, pp_part)


def kernel(track_point_feats, p2t_src, p2t_dst, W, b, ln_gamma, ln_beta):
    x = track_point_feats
    h = _mlp(x, W, b, ln_gamma, ln_beta)
    zacc = jnp.zeros((N_TRACKS * HALF,), jnp.float32)
    tp_part = _seg_max(h, p2t_src, p2t_dst, zacc)
    tp_part = tp_part.reshape(NC, 2, N_TRACKS, HALF)
    track_pool = _tpmax(tp_part)
    zpp = jnp.zeros((NS, PP_SLICE, D), jnp.float32)
    pp_part = _seg_sum(track_pool, p2t_src, p2t_dst, zpp)
    pp_part = pp_part.reshape(NC, N_POINTS, D)
    out_features = _outk(x, pp_part)
    return out_features, track_pool


# speedup vs baseline: 5.1944x; 2.0166x over previous
import functools

import jax
import jax.numpy as jnp
from jax import lax
from jax.experimental import pallas as pl
from jax.experimental.pallas import tpu as pltpu
from jax.experimental.pallas import tpu_sc as plsc

N_POINTS = 10000
N_TRACKS = 1000
N_EDGES = 320000
D = 128
HALF = 64
LN_EPS = 1e-5

NC = 2
NS = 16

_SC_MESH = plsc.VectorSubcoreMesh(core_axis_name="c", subcore_axis_name="s")


def _mlp_body(x_ref, w_ref, b_ref, g_ref, be_ref, o_ref):
    h = jnp.dot(x_ref[...], w_ref[...], preferred_element_type=jnp.float32,
                precision=jax.lax.Precision.HIGHEST) + b_ref[...]
    mu = jnp.mean(h, axis=-1, keepdims=True)
    var = jnp.mean((h - mu) ** 2, axis=-1, keepdims=True)
    hn = (h - mu) * jax.lax.rsqrt(var + LN_EPS)
    o_ref[...] = jnp.maximum(hn * g_ref[...] + be_ref[...], 0.0)


def _mlp(x, W, b, gamma, beta):
    return pl.pallas_call(
        _mlp_body,
        out_shape=jax.ShapeDtypeStruct((N_POINTS, D), jnp.float32),
        grid=(5,),
        in_specs=[
            pl.BlockSpec((2000, D), lambda i: (i, 0)),
            pl.BlockSpec((D, D), lambda i: (0, 0)),
            pl.BlockSpec((1, D), lambda i: (0, 0)),
            pl.BlockSpec((1, D), lambda i: (0, 0)),
            pl.BlockSpec((1, D), lambda i: (0, 0)),
        ],
        out_specs=pl.BlockSpec((2000, D), lambda i: (i, 0)),
    )(x, W, b.reshape(1, D), gamma.reshape(1, D), beta.reshape(1, D))


P1_CHUNK = 80
P1_GROUPS = 8
P1_EPT = N_EDGES // (NC * P1_GROUPS)
P1_NCHUNK = P1_EPT // P1_CHUNK
TRK_SLICE = N_TRACKS // P1_GROUPS


def _seg_max(h, src, dst, zacc):
    @functools.partial(
        pl.kernel,
        out_type=jax.ShapeDtypeStruct((NC * 2 * N_TRACKS * HALF,),
                                      jnp.float32),
        mesh=_SC_MESH,
        scratch_types=[
            pltpu.VMEM((N_TRACKS * HALF,), jnp.float32),
            pltpu.VMEM((2, P1_CHUNK), jnp.int32),
            pltpu.VMEM((2, P1_CHUNK), jnp.int32),
            pltpu.VMEM((2, P1_CHUNK), jnp.int32),
            pltpu.VMEM((2, P1_CHUNK), jnp.int32),
            pltpu.VMEM((P1_CHUNK, D), jnp.float32),
            pltpu.VMEM((P1_CHUNK, D), jnp.float32),
            pltpu.VMEM((N_TRACKS * HALF // 8,), jnp.float32),
            pltpu.VMEM_SHARED((8 * N_TRACKS * HALF,), jnp.float32),
            pltpu.SemaphoreType.DMA,
            pltpu.SemaphoreType.DMA,
            pltpu.SemaphoreType.DMA,
            pltpu.SemaphoreType.DMA,
        ],
    )
    def body(h_hbm, src_hbm, dst_hbm, z_hbm, out_hbm,
             acc, sbufE, sbufO, dbufE, dbufO, rbufE, rbufO, tmp, stage,
             isemE, isemO, gsemE, gsemO):
        k = lax.axis_index("c")
        s = lax.axis_index("s")
        half = s // P1_GROUPS
        grp = s % P1_GROUPS
        base = k * (N_EDGES // NC) + grp * P1_EPT
        NP = P1_NCHUNK // 2

        pltpu.sync_copy(z_hbm, acc)

        def idx_pair(c, sb, db, slot, sem):
            off = base + c * P1_CHUNK
            return (pltpu.make_async_copy(
                        src_hbm.at[pl.ds(off, P1_CHUNK)], sb.at[slot], sem),
                    pltpu.make_async_copy(
                        dst_hbm.at[pl.ds(off, P1_CHUNK)], db.at[slot], sem))

        def gath(sb, slot, rb, sem):
            return pltpu.make_async_copy(h_hbm.at[sb.at[slot]], rb, sem)

        def run(coff):
            def compute(rb, db, slot):
                @pl.loop(0, P1_CHUNK // 16)
                def _edge16(i):
                    dvec = db[slot, pl.ds(i * 16, 16)]
                    for jj in range(16):
                        dd = dvec[jj]
                        j = i * 16 + jj
                        for cc in range(HALF // 16):
                            asl = pl.ds(dd * HALF + cc * 16, 16)
                            rsl = pl.ds(coff + cc * 16, 16)
                            acc[asl] = jnp.maximum(acc[asl], rb[j, rsl])

            a0, b0 = idx_pair(0, sbufE, dbufE, 0, isemE)
            a0.start(); b0.start()
            a1, b1 = idx_pair(1, sbufO, dbufO, 0, isemO)
            a1.start(); b1.start()
            a0.wait(); b0.wait()
            gath(sbufE, 0, rbufE, gsemE).start()

            @pl.loop(0, NP)
            def _pair(i):
                cur = i % 2
                nxt = (i + 1) % 2
                more = i < NP - 1

                @pl.when(more)
                def _():
                    a, bb = idx_pair(2 * i + 2, sbufE, dbufE, nxt, isemE)
                    a.start(); bb.start()

                aw, bw = idx_pair(2 * i + 1, sbufO, dbufO, cur, isemO)
                aw.wait(); bw.wait()
                gath(sbufO, cur, rbufO, gsemO).start()

                @pl.when(more)
                def _():
                    a, bb = idx_pair(2 * i + 3, sbufO, dbufO, nxt, isemO)
                    a.start(); bb.start()

                gath(sbufE, cur, rbufE, gsemE).wait()
                compute(rbufE, dbufE, cur)

                @pl.when(more)
                def _():
                    a, bb = idx_pair(2 * i + 2, sbufE, dbufE, nxt, isemE)
                    a.wait(); bb.wait()
                    gath(sbufE, nxt, rbufE, gsemE).start()

                gath(sbufO, cur, rbufO, gsemO).wait()
                compute(rbufO, dbufO, cur)

        @pl.when(half == 0)
        def _lo():
            run(0)

        @pl.when(half == 1)
        def _hi():
            run(HALF)

        FLAT = N_TRACKS * HALF
        CNK = FLAT // 8
        for m in (4, 2, 1):
            @pl.when(jnp.logical_and(grp >= m, grp < 2 * m))
            def _stage():
                slot = half * 4 + (grp - m)
                pltpu.sync_copy(acc, stage.at[pl.ds(slot * FLAT, FLAT)])

            plsc.subcore_barrier()

            @pl.when(grp < m)
            def _merge():
                slot = half * 4 + grp

                @pl.loop(0, 8)
                def _cnk(c):
                    pltpu.sync_copy(
                        stage.at[pl.ds(slot * FLAT + c * CNK, CNK)], tmp)

                    @pl.loop(0, CNK // 16)
                    def _vec(v):
                        asl = pl.ds(c * CNK + v * 16, 16)
                        tsl = pl.ds(v * 16, 16)
                        acc[asl] = jnp.maximum(acc[asl], tmp[tsl])

            plsc.subcore_barrier()

        @pl.when(grp == 0)
        def _writeout():
            pltpu.sync_copy(
                acc, out_hbm.at[pl.ds((k * 2 + half) * FLAT, FLAT)])

    return body(h, src, dst, zacc)


def _tpmax_body(p_ref, o_ref):
    m = jnp.maximum(p_ref[0], p_ref[1])
    o_ref[...] = jnp.concatenate([m[0], m[1]], axis=1)


def _tpmax(tp_part):
    return pl.pallas_call(
        _tpmax_body,
        out_shape=jax.ShapeDtypeStruct((N_TRACKS, D), jnp.float32),
        grid=(1,),
        in_specs=[pl.BlockSpec((NC, 2, N_TRACKS, HALF),
                               lambda i: (0, 0, 0, 0))],
        out_specs=pl.BlockSpec((N_TRACKS, D), lambda i: (0, 0)),
    )(tp_part)


P2_CHUNK = 80
P2_EPT = N_EDGES // (NC * NS)
P2_NCHUNK = P2_EPT // P2_CHUNK
PP_SLICE = N_POINTS // NS


def _seg_sum(tp, src, dst, zpp):
    @functools.partial(
        pl.kernel,
        out_type=jax.ShapeDtypeStruct((NC, NS, PP_SLICE, D), jnp.float32),
        mesh=_SC_MESH,
        scratch_types=[
            pltpu.VMEM((2, P2_CHUNK), jnp.int32),
            pltpu.VMEM((2, P2_CHUNK), jnp.int32),
            pltpu.VMEM((2, P2_CHUNK), jnp.int32),
            pltpu.VMEM((2, P2_CHUNK), jnp.int32),
            pltpu.VMEM((2, P2_CHUNK, D), jnp.float32),
            pltpu.VMEM((P2_CHUNK, D), jnp.float32),
            pltpu.VMEM_SHARED((N_POINTS, D), jnp.float32),
            pltpu.SemaphoreType.DMA,
            pltpu.SemaphoreType.DMA,
            pltpu.SemaphoreType.DMA,
            pltpu.SemaphoreType.DMA,
            pltpu.SemaphoreType.DMA,
            pltpu.SemaphoreType.DMA,
        ],
    )
    def body(tp_hbm, src_hbm, dst_hbm, z_hbm, out_hbm,
             sbufE, sbufO, dbufE, dbufO, rbufE, rbufO, pp,
             isemE, isemO, gsemE, gsemO, ssemE, ssemO):
        k = lax.axis_index("c")
        s = lax.axis_index("s")
        base = k * (N_EDGES // NC) + s * P2_EPT
        rowbase = s * PP_SLICE
        NP = P2_NCHUNK // 2

        pltpu.sync_copy(z_hbm.at[s], pp.at[pl.ds(rowbase, PP_SLICE)])
        plsc.subcore_barrier()

        def idx_pair(c, sb, db, slot, sem):
            off = base + c * P2_CHUNK
            return (pltpu.make_async_copy(
                        src_hbm.at[pl.ds(off, P2_CHUNK)], sb.at[slot], sem),
                    pltpu.make_async_copy(
                        dst_hbm.at[pl.ds(off, P2_CHUNK)], db.at[slot], sem))

        def gath(db, slot, rb, sem):
            return pltpu.make_async_copy(tp_hbm.at[db.at[slot]], rb, sem)

        def scat_start(rb, sb, slot, sem):
            pltpu.async_copy(rb, pp.at[sb.at[slot]], sem, add=True)

        def scat_wait(rb, sb, slot, sem):
            pltpu.make_async_copy(rb, pp.at[sb.at[slot]], sem).wait()

        a0, b0 = idx_pair(0, sbufE, dbufE, 0, isemE)
        a0.start(); b0.start()
        a1, b1 = idx_pair(1, sbufO, dbufO, 0, isemO)
        a1.start(); b1.start()
        a0.wait(); b0.wait()
        gath(dbufE, 0, rbufE.at[0], gsemE).start()

        @pl.loop(0, NP)
        def _pair(i):
            cur = i % 2
            nxt = (i + 1) % 2

            @pl.when(i > 0)
            def _():
                scat_wait(rbufE.at[cur], sbufE, nxt, ssemE)

            a, bb = idx_pair(2 * i + 2, sbufE, dbufE, nxt, isemE)
            a.start(); bb.start()

            aw, bw = idx_pair(2 * i + 1, sbufO, dbufO, cur, isemO)
            aw.wait(); bw.wait()

            @pl.when(i > 0)
            def _():
                scat_wait(rbufO, sbufO, nxt, ssemO)

            gath(dbufO, cur, rbufO, gsemO).start()

            @pl.when(i < NP - 1)
            def _():
                a2, b2 = idx_pair(2 * i + 3, sbufO, dbufO, nxt, isemO)
                a2.start(); b2.start()

            gath(dbufE, cur, rbufE.at[cur], gsemE).wait()
            scat_start(rbufE.at[cur], sbufE, cur, ssemE)

            a, bb = idx_pair(2 * i + 2, sbufE, dbufE, nxt, isemE)
            a.wait(); bb.wait()
            gath(dbufE, nxt, rbufE.at[nxt], gsemE).start()

            gath(dbufO, cur, rbufO, gsemO).wait()
            scat_start(rbufO, sbufO, cur, ssemO)

        last = NP % 2
        scat_wait(rbufE.at[1 - last], sbufE, 1 - last, ssemE)
        gath(dbufE, last, rbufE.at[last], gsemE).wait()
        scat_start(rbufE.at[last], sbufE, last, ssemE)
        scat_wait(rbufO, sbufO, 1 - last, ssemO)
        scat_wait(rbufE.at[last], sbufE, last, ssemE)

        plsc.subcore_barrier()
        pltpu.sync_copy(pp.at[pl.ds(rowbase, PP_SLICE)], out_hbm.at[k, s])

    return body(tp, src, dst, zpp)


def _out_body(x_ref, pp_ref, o_ref):
    o_ref[...] = jnp.concatenate([x_ref[...], pp_ref[0] + pp_ref[1]], axis=1)


def _outk(x, pp_part):
    return pl.pallas_call(
        _out_body,
        out_shape=jax.ShapeDtypeStruct((N_POINTS, 2 * D), jnp.float32),
        grid=(5,),
        in_specs=[
            pl.BlockSpec((2000, D), lambda i: (i, 0)),
            pl.BlockSpec((NC, 2000, D), lambda i: (0, i, 0)),
        ],
        out_specs=pl.BlockSpec((2000, 2 * D), lambda i: (i, 0)),
    )(x, pp_part)


def kernel(track_point_feats, p2t_src, p2t_dst, W, b, ln_gamma, ln_beta):
    x = track_point_feats
    h = _mlp(x, W, b, ln_gamma, ln_beta)
    zacc = jnp.zeros((N_TRACKS * HALF,), jnp.float32)
    tp_part = _seg_max(h, p2t_src, p2t_dst, zacc)
    tp_part = tp_part.reshape(NC, 2, N_TRACKS, HALF)
    track_pool = _tpmax(tp_part)
    zpp = jnp.zeros((NS, PP_SLICE, D), jnp.float32)
    pp_part = _seg_sum(track_pool, p2t_src, p2t_dst, zpp)
    pp_part = pp_part.reshape(NC, N_POINTS, D)
    out_features = _outk(x, pp_part)
    return out_features, track_pool
